# baseline (speedup 1.0000x reference)
import functools

import jax
import jax.numpy as jnp
from jax import lax
from jax.experimental import pallas as pl
from jax.experimental.pallas import tpu as pltpu
from jax.experimental.pallas import tpu_sc as plsc

_EPS = 1e-5


def _softplus(x):
    return jnp.maximum(x, 0.0) + jnp.log1p(jnp.exp(-jnp.abs(x)))


def _sigmoid(x):
    return 1.0 / (1.0 + jnp.exp(-x))


def _sc_gather(table, idx_flat, chunk=200):
    n_rows, d = table.shape
    b = idx_flat.shape[0]
    info = plsc.get_sparse_core_info()
    nw = info.num_cores * info.num_subcores
    per_w = b // nw
    assert per_w * nw == b and per_w % (2 * chunk) == 0 and chunk % 8 == 0
    n_half = per_w // chunk // 2
    mesh = plsc.VectorSubcoreMesh(core_axis_name="c", subcore_axis_name="s")

    @functools.partial(
        pl.kernel,
        mesh=mesh,
        out_type=jax.ShapeDtypeStruct((b, d), table.dtype),
        compiler_params=pltpu.CompilerParams(use_tc_tiling_on_sc=True),
        scratch_types=[
            pltpu.VMEM((chunk,), jnp.int32),
            pltpu.VMEM((chunk,), jnp.int32),
            pltpu.VMEM((chunk, d), table.dtype),
            pltpu.VMEM((chunk, d), table.dtype),
            pltpu.SemaphoreType.DMA,
            pltpu.SemaphoreType.DMA,
            pltpu.SemaphoreType.DMA,
        ],
    )
    def k(table_hbm, idx_hbm, out_hbm, idx_v0, idx_v1, rows_v0, rows_v1,
          sem_g, sem_o0, sem_o1):
        wid = lax.axis_index("s") * info.num_cores + lax.axis_index("c")
        base = pl.multiple_of(wid * per_w, 8)
        idx_v = (idx_v0, idx_v1)
        rows_v = (rows_v0, rows_v1)
        sem_o = (sem_o0, sem_o1)

        def body(j, _):
            for sl in range(2):
                ci = 2 * j + sl
                off = pl.multiple_of(base + ci * chunk, 8)

                @pl.when(j > 0)
                def _drain():
                    pltpu.make_async_copy(
                        rows_v[sl], out_hbm.at[pl.ds(off, chunk)],
                        sem_o[sl]).wait()

                pltpu.sync_copy(idx_hbm.at[pl.ds(off, chunk)], idx_v[sl])
                pltpu.async_copy(table_hbm.at[idx_v[sl]], rows_v[sl],
                                 sem_g).wait()
                pltpu.async_copy(rows_v[sl],
                                 out_hbm.at[pl.ds(off, chunk)], sem_o[sl])
            return ()

        lax.fori_loop(0, n_half, body, (), unroll=False)
        for sl in range(2):
            pltpu.make_async_copy(rows_v[sl],
                                  out_hbm.at[pl.ds(base, chunk)],
                                  sem_o[sl]).wait()

    return k(table, idx_flat)


def _center_body(atom_ref, wc_ref, bf_ref, out_ref):
    out_ref[...] = (
        jnp.dot(atom_ref[...], wc_ref[...], preferred_element_type=jnp.float32)
        + bf_ref[...]
    )


def _edge_preact(g, q, wg, wn):
    return (
        jnp.dot(g.astype(jnp.bfloat16), wg, preferred_element_type=jnp.float32)
        + jnp.dot(q, wn, preferred_element_type=jnp.float32)
    )


def _stats_body(mlast, g_ref, q_ref, a_ref, wg_ref, wn_ref, stats_ref):
    nb = pl.program_id(0)
    mm = pl.program_id(1)

    @pl.when((nb == 0) & (mm == 0))
    def _init():
        stats_ref[...] = jnp.zeros_like(stats_ref)

    q = q_ref[...].reshape(q_ref.shape[1], q_ref.shape[2])
    x2 = _edge_preact(g_ref[...], q, wg_ref[...], wn_ref[...])
    hf = x2.shape[1]
    s = jnp.sum(x2, axis=0)
    sq = jnp.sum(x2 * x2, axis=0)
    a = a_ref[...]
    cross = jnp.sum(a * x2, axis=0)
    stats_ref[0:1, :] += s.reshape(1, hf)
    stats_ref[1:2, :] += (sq + 2.0 * cross).reshape(1, hf)

    @pl.when(mm == 0)
    def _atom_terms():
        cm = float(mlast + 1)
        stats_ref[0:1, :] += cm * jnp.sum(a, axis=0).reshape(1, hf)
        stats_ref[1:2, :] += cm * jnp.sum(a * a, axis=0).reshape(1, hf)


def _apply_body(count, mlast, g_ref, q_ref, a_ref, w_ref, wg_ref, wn_ref,
                stats_ref, g1_ref, b1_ref, s_out_ref, st2_ref,
                a2_scr, wgs_scr, wns_scr):
    nb = pl.program_id(0)
    mm = pl.program_id(1)

    @pl.when((nb == 0) & (mm == 0))
    def _init_stats():
        st2_ref[...] = jnp.zeros_like(st2_ref)

    mu = stats_ref[0:1, :] / count
    ex2 = stats_ref[1:2, :] / count
    var = ex2 - mu * mu
    inv = lax.rsqrt(var + _EPS)
    scale = g1_ref[...] * inv
    shift = b1_ref[...] - mu * scale

    @pl.when((nb == 0) & (mm == 0))
    def _scale_weights():
        wgs_scr[...] = (wg_ref[...] * scale.astype(jnp.bfloat16))
        wns_scr[...] = (wn_ref[...] * scale.astype(jnp.bfloat16))

    @pl.when(mm == 0)
    def _center():
        a2_scr[...] = a_ref[...] * scale + shift

    q = q_ref[...].reshape(q_ref.shape[1], q_ref.shape[2])
    y = _edge_preact(g_ref[...], q, wgs_scr[...], wns_scr[...]) + a2_scr[...]
    af = y.shape[1] // 2
    f = _sigmoid(y[:, :af])
    c = _softplus(y[:, af:])
    w = w_ref[...]
    contrib = f * c * (w * w)

    @pl.when(mm == 0)
    def _first():
        s_out_ref[...] = contrib

    @pl.when(mm > 0)
    def _accum():
        s_out_ref[...] += contrib

    @pl.when(mm == mlast)
    def _bn2_stats():
        s_blk = s_out_ref[...]
        st2_ref[0:1, :] += jnp.sum(s_blk, axis=0).reshape(1, af)
        st2_ref[1:2, :] += jnp.sum(s_blk * s_blk, axis=0).reshape(1, af)


def _final_body(count, s_ref, atom_ref, st2_ref, g2_ref, b2_ref,
                wp_ref, bp_ref, out_ref):
    mu = st2_ref[0:1, :] / count
    ex2 = st2_ref[1:2, :] / count
    var = ex2 - mu * mu
    inv = lax.rsqrt(var + _EPS)
    scale = g2_ref[...] * inv
    shift = b2_ref[...] - mu * scale
    h = _softplus(atom_ref[...] + s_ref[...] * scale + shift)
    out_ref[...] = (
        jnp.dot(h, wp_ref[...], preferred_element_type=jnp.float32)
        + bp_ref[...]
    )


def _tc_pipeline(atom_fea, g0t, nbr_t, bond_t, W_full, b_full,
                 g1, b1, g2, b2, W_proj, b_proj):
    n, af = atom_fea.shape
    nm = g0t.shape[0]
    m = nm // n
    nf = nbr_t.shape[2]
    hf = 2 * af

    wc = W_full[:af]
    wg = W_full[af:2 * af].astype(jnp.bfloat16)
    wn = W_full[2 * af:].astype(jnp.bfloat16)

    a_center = pl.pallas_call(
        _center_body,
        out_shape=jax.ShapeDtypeStruct((n, hf), jnp.float32),
    )(atom_fea, wc, b_full.reshape(1, hf))

    bn = 2000
    nblk = n // bn
    grid = (nblk, m)
    full = lambda shp: pl.BlockSpec(shp, lambda i, j: (0,) * len(shp))
    g_spec = pl.BlockSpec((bn, af), lambda i, j: (j * nblk + i, 0))
    q_spec = pl.BlockSpec((1, bn, nf), lambda i, j: (j, i, 0))
    a_spec = pl.BlockSpec((bn, hf), lambda i, j: (i, 0))

    stats = pl.pallas_call(
        functools.partial(_stats_body, m - 1),
        grid=grid,
        in_specs=[g_spec, q_spec, a_spec, full((af, hf)), full((nf, hf))],
        out_specs=pl.BlockSpec((8, hf), lambda i, j: (0, 0)),
        out_shape=jax.ShapeDtypeStruct((8, hf), jnp.float32),
    )(g0t, nbr_t, a_center, wg, wn)

    s_sum, st2 = pl.pallas_call(
        functools.partial(_apply_body, float(nm), m - 1),
        grid=grid,
        in_specs=[
            g_spec,
            q_spec,
            a_spec,
            pl.BlockSpec((bn, 1), lambda i, j: (j * nblk + i, 0)),
            full((af, hf)),
            full((nf, hf)),
            full((8, hf)),
            full((1, hf)),
            full((1, hf)),
        ],
        out_specs=[
            pl.BlockSpec((bn, af), lambda i, j: (i, 0)),
            pl.BlockSpec((8, af), lambda i, j: (0, 0)),
        ],
        out_shape=[
            jax.ShapeDtypeStruct((n, af), jnp.float32),
            jax.ShapeDtypeStruct((8, af), jnp.float32),
        ],
        scratch_shapes=[
            pltpu.VMEM((bn, hf), jnp.float32),
            pltpu.VMEM((af, hf), jnp.bfloat16),
            pltpu.VMEM((nf, hf), jnp.bfloat16),
        ],
    )(g0t, nbr_t, a_center, bond_t, wg, wn, stats,
      g1.reshape(1, hf), b1.reshape(1, hf))

    ba2 = 2000
    atom_out = pl.pallas_call(
        functools.partial(_final_body, float(n)),
        grid=(n // ba2,),
        in_specs=[
            pl.BlockSpec((ba2, af), lambda i: (i, 0)),
            pl.BlockSpec((ba2, af), lambda i: (i, 0)),
            pl.BlockSpec((8, af), lambda i: (0, 0)),
            pl.BlockSpec((1, af), lambda i: (0, 0)),
            pl.BlockSpec((1, af), lambda i: (0, 0)),
            pl.BlockSpec((af, af), lambda i: (0, 0)),
            pl.BlockSpec((1, af), lambda i: (0, 0)),
        ],
        out_specs=pl.BlockSpec((ba2, af), lambda i: (i, 0)),
        out_shape=jax.ShapeDtypeStruct((n, af), jnp.float32),
    )(s_sum, atom_fea, st2, g2.reshape(1, af), b2.reshape(1, af),
      W_proj, b_proj.reshape(1, af))

    return atom_out


def kernel(atom_fea, nbr_fea, nbr_fea_idx, bond_weights_ag,
           W_full, b_full, g1, b1, g2, b2, W_proj, b_proj):
    n, m = nbr_fea_idx.shape
    nf = nbr_fea.shape[2]
    idx_t = nbr_fea_idx.T.reshape(n * m).astype(jnp.int32)
    g0t = _sc_gather(atom_fea, idx_t)
    nbr_t = jnp.transpose(nbr_fea, (1, 0, 2)).astype(jnp.bfloat16)
    bond_t = bond_weights_ag.T.reshape(n * m, 1)
    atom_out = _tc_pipeline(atom_fea, g0t, nbr_t, bond_t,
                            W_full, b_full, g1, b1, g2, b2, W_proj, b_proj)
    return atom_out, nbr_fea


# speedup vs baseline: 1.3492x; 1.3492x over previous
import functools

import jax
import jax.numpy as jnp
from jax import lax
from jax.experimental import pallas as pl
from jax.experimental.pallas import tpu as pltpu
from jax.experimental.pallas import tpu_sc as plsc

_EPS = 1e-5


def _softplus(x):
    return jnp.maximum(x, 0.0) + jnp.log1p(jnp.exp(-jnp.abs(x)))


def _sigmoid(x):
    return 0.5 * jnp.tanh(0.5 * x) + 0.5


def _sc_gather(table, idx_flat, chunk=400):
    n_rows, d = table.shape
    b = idx_flat.shape[0]
    info = plsc.get_sparse_core_info()
    nw = info.num_cores * info.num_subcores
    per_w = b // nw
    assert per_w * nw == b and per_w % chunk == 0 and chunk % 8 == 0
    n_chunks = per_w // chunk
    n_pairs = n_chunks // 2
    tail = n_chunks % 2
    mesh = plsc.VectorSubcoreMesh(core_axis_name="c", subcore_axis_name="s")

    @functools.partial(
        pl.kernel,
        mesh=mesh,
        out_type=jax.ShapeDtypeStruct((b, d), table.dtype),
        compiler_params=pltpu.CompilerParams(use_tc_tiling_on_sc=True),
        scratch_types=[
            pltpu.VMEM((chunk,), jnp.int32),
            pltpu.VMEM((chunk,), jnp.int32),
            pltpu.VMEM((chunk, d), table.dtype),
            pltpu.VMEM((chunk, d), table.dtype),
            pltpu.SemaphoreType.DMA,
            pltpu.SemaphoreType.DMA,
            pltpu.SemaphoreType.DMA,
        ],
    )
    def k(table_hbm, idx_hbm, out_hbm, idx_v0, idx_v1, rows_v0, rows_v1,
          sem_g, sem_o0, sem_o1):
        wid = lax.axis_index("s") * info.num_cores + lax.axis_index("c")
        base = pl.multiple_of(wid * per_w, 8)
        idx_v = (idx_v0, idx_v1)
        rows_v = (rows_v0, rows_v1)
        sem_o = (sem_o0, sem_o1)

        def chunk_step(ci, sl, guard):
            off = pl.multiple_of(base + ci * chunk, 8)

            @pl.when(guard)
            def _drain():
                pltpu.make_async_copy(
                    rows_v[sl], out_hbm.at[pl.ds(off, chunk)],
                    sem_o[sl]).wait()

            pltpu.sync_copy(idx_hbm.at[pl.ds(off, chunk)], idx_v[sl])
            pltpu.async_copy(table_hbm.at[idx_v[sl]], rows_v[sl],
                             sem_g).wait()
            pltpu.async_copy(rows_v[sl],
                             out_hbm.at[pl.ds(off, chunk)], sem_o[sl])

        def body(j, _):
            for sl in range(2):
                chunk_step(2 * j + sl, sl, j > 0)
            return ()

        lax.fori_loop(0, n_pairs, body, (), unroll=False)
        if tail:
            chunk_step(2 * n_pairs, 0, n_pairs > 0)
        for sl in range(2):
            if (n_pairs > 0) or (tail and sl == 0):
                pltpu.make_async_copy(rows_v[sl],
                                      out_hbm.at[pl.ds(base, chunk)],
                                      sem_o[sl]).wait()

    return k(table, idx_flat)


def _center_body(atom_ref, wc_ref, bf_ref, out_ref):
    out_ref[...] = (
        jnp.dot(atom_ref[...], wc_ref[...], preferred_element_type=jnp.float32)
        + bf_ref[...]
    )


def _edge_preact(g, q, wg, wn):
    return (
        jnp.dot(g.astype(jnp.bfloat16), wg, preferred_element_type=jnp.float32)
        + jnp.dot(q, wn, preferred_element_type=jnp.float32)
    )


def _stats_body(g_ref, q_ref, a_ref, wg_ref, wn_ref, stats_ref):
    step = pl.program_id(0)

    @pl.when(step == 0)
    def _init():
        stats_ref[...] = jnp.zeros_like(stats_ref)

    a = a_ref[...]
    q = q_ref[...].astype(jnp.bfloat16)
    x2 = _edge_preact(g_ref[...], q, wg_ref[...], wn_ref[...])
    ba, hf = a.shape
    m = x2.shape[0] // ba
    t = jnp.sum(x2.reshape(ba, m, hf), axis=1)
    s = jnp.sum(x2, axis=0) + m * jnp.sum(a, axis=0)
    sq = (
        jnp.sum(x2 * x2, axis=0)
        + 2.0 * jnp.sum(a * t, axis=0)
        + m * jnp.sum(a * a, axis=0)
    )
    stats_ref[0:1, :] += s.reshape(1, hf)
    stats_ref[1:2, :] += sq.reshape(1, hf)


def _apply_body(count, g_ref, q_ref, a_ref, w_ref, wg_ref, wn_ref,
                stats_ref, g1_ref, b1_ref, s_out_ref, st2_ref,
                wgs_scr, wns_scr):
    step = pl.program_id(0)

    @pl.when(step == 0)
    def _init_stats():
        st2_ref[...] = jnp.zeros_like(st2_ref)

    mu = stats_ref[0:1, :] / count
    ex2 = stats_ref[1:2, :] / count
    var = ex2 - mu * mu
    inv = lax.rsqrt(var + _EPS)
    scale = g1_ref[...] * inv
    shift = b1_ref[...] - mu * scale

    @pl.when(step == 0)
    def _scale_weights():
        wgs_scr[...] = (wg_ref[...] * scale.astype(jnp.bfloat16))
        wns_scr[...] = (wn_ref[...] * scale.astype(jnp.bfloat16))

    q = q_ref[...].astype(jnp.bfloat16)
    y2 = _edge_preact(g_ref[...], q, wgs_scr[...], wns_scr[...])
    a = a_ref[...]
    ba, hf = a.shape
    m = y2.shape[0] // ba
    af = hf // 2
    a2 = a * scale + shift
    y3 = y2.reshape(ba, m, hf) + a2[:, None, :]
    f = _sigmoid(y3[:, :, :af])
    c = _softplus(y3[:, :, af:])
    w = w_ref[...]
    prod = f * c * (w * w)[:, :, None]
    s_blk = jnp.sum(prod, axis=1)
    s_out_ref[...] = s_blk
    st2_ref[0:1, :] += jnp.sum(s_blk, axis=0).reshape(1, af)
    st2_ref[1:2, :] += jnp.sum(s_blk * s_blk, axis=0).reshape(1, af)


def _final_body(count, s_ref, atom_ref, st2_ref, g2_ref, b2_ref,
                wp_ref, bp_ref, out_ref):
    mu = st2_ref[0:1, :] / count
    ex2 = st2_ref[1:2, :] / count
    var = ex2 - mu * mu
    inv = lax.rsqrt(var + _EPS)
    scale = g2_ref[...] * inv
    shift = b2_ref[...] - mu * scale
    h = _softplus(atom_ref[...] + s_ref[...] * scale + shift)
    out_ref[...] = (
        jnp.dot(h, wp_ref[...], preferred_element_type=jnp.float32)
        + bp_ref[...]
    )


def _tc_pipeline(atom_fea, g0, nbr_flat, bond_w, W_full, b_full,
                 g1, b1, g2, b2, W_proj, b_proj):
    n, af = atom_fea.shape
    nm = g0.shape[0]
    m = nm // n
    nf = nbr_flat.shape[1]
    hf = 2 * af

    wc = W_full[:af]
    wg = W_full[af:2 * af].astype(jnp.bfloat16)
    wn = W_full[2 * af:].astype(jnp.bfloat16)

    a_center = pl.pallas_call(
        _center_body,
        out_shape=jax.ShapeDtypeStruct((n, hf), jnp.float32),
    )(atom_fea, wc, b_full.reshape(1, hf))

    ba = 200
    be = ba * m
    nsteps = n // ba
    full = lambda shp: pl.BlockSpec(shp, lambda i: (0,) * len(shp))

    stats = pl.pallas_call(
        _stats_body,
        grid=(nsteps,),
        in_specs=[
            pl.BlockSpec((be, af), lambda i: (i, 0)),
            pl.BlockSpec((be, nf), lambda i: (i, 0)),
            pl.BlockSpec((ba, hf), lambda i: (i, 0)),
            full((af, hf)),
            full((nf, hf)),
        ],
        out_specs=pl.BlockSpec((8, hf), lambda i: (0, 0)),
        out_shape=jax.ShapeDtypeStruct((8, hf), jnp.float32),
    )(g0, nbr_flat, a_center, wg, wn)

    s_sum, st2 = pl.pallas_call(
        functools.partial(_apply_body, float(nm)),
        grid=(nsteps,),
        in_specs=[
            pl.BlockSpec((be, af), lambda i: (i, 0)),
            pl.BlockSpec((be, nf), lambda i: (i, 0)),
            pl.BlockSpec((ba, hf), lambda i: (i, 0)),
            pl.BlockSpec((ba, m), lambda i: (i, 0)),
            full((af, hf)),
            full((nf, hf)),
            full((8, hf)),
            full((1, hf)),
            full((1, hf)),
        ],
        out_specs=[
            pl.BlockSpec((ba, af), lambda i: (i, 0)),
            pl.BlockSpec((8, af), lambda i: (0, 0)),
        ],
        out_shape=[
            jax.ShapeDtypeStruct((n, af), jnp.float32),
            jax.ShapeDtypeStruct((8, af), jnp.float32),
        ],
        scratch_shapes=[
            pltpu.VMEM((af, hf), jnp.bfloat16),
            pltpu.VMEM((nf, hf), jnp.bfloat16),
        ],
    )(g0, nbr_flat, a_center, bond_w, wg, wn, stats,
      g1.reshape(1, hf), b1.reshape(1, hf))

    ba2 = 2000
    atom_out = pl.pallas_call(
        functools.partial(_final_body, float(n)),
        grid=(n // ba2,),
        in_specs=[
            pl.BlockSpec((ba2, af), lambda i: (i, 0)),
            pl.BlockSpec((ba2, af), lambda i: (i, 0)),
            pl.BlockSpec((8, af), lambda i: (0, 0)),
            pl.BlockSpec((1, af), lambda i: (0, 0)),
            pl.BlockSpec((1, af), lambda i: (0, 0)),
            pl.BlockSpec((af, af), lambda i: (0, 0)),
            pl.BlockSpec((1, af), lambda i: (0, 0)),
        ],
        out_specs=pl.BlockSpec((ba2, af), lambda i: (i, 0)),
        out_shape=jax.ShapeDtypeStruct((n, af), jnp.float32),
    )(s_sum, atom_fea, st2, g2.reshape(1, af), b2.reshape(1, af),
      W_proj, b_proj.reshape(1, af))

    return atom_out


def kernel(atom_fea, nbr_fea, nbr_fea_idx, bond_weights_ag,
           W_full, b_full, g1, b1, g2, b2, W_proj, b_proj):
    n, m = nbr_fea_idx.shape
    nf = nbr_fea.shape[2]
    idx_flat = nbr_fea_idx.reshape(n * m).astype(jnp.int32)
    g0 = _sc_gather(atom_fea, idx_flat)
    nbr_flat = nbr_fea.reshape(n * m, nf)
    atom_out = _tc_pipeline(atom_fea, g0, nbr_flat, bond_weights_ag,
                            W_full, b_full, g1, b1, g2, b2, W_proj, b_proj)
    return atom_out, nbr_fea


# speedup vs baseline: 1.3539x; 1.0035x over previous
import functools

import jax
import jax.numpy as jnp
from jax import lax
from jax.experimental import pallas as pl
from jax.experimental.pallas import tpu as pltpu
from jax.experimental.pallas import tpu_sc as plsc

_EPS = 1e-5


def _softplus(x):
    return jnp.maximum(x, 0.0) + jnp.log1p(jnp.exp(-jnp.abs(x)))


def _sigmoid(x):
    return 0.5 * jnp.tanh(0.5 * x) + 0.5


def _sc_gather(table, idx_flat, chunk=400):
    n_rows, d = table.shape
    b = idx_flat.shape[0]
    info = plsc.get_sparse_core_info()
    nw = info.num_cores * info.num_subcores
    per_w = b // nw
    while per_w % chunk != 0:
        chunk //= 2
    assert per_w * nw == b and per_w % chunk == 0 and chunk % 8 == 0
    n_chunks = per_w // chunk
    n_pairs = n_chunks // 2
    tail = n_chunks % 2
    mesh = plsc.VectorSubcoreMesh(core_axis_name="c", subcore_axis_name="s")

    @functools.partial(
        pl.kernel,
        mesh=mesh,
        out_type=jax.ShapeDtypeStruct((b, d), table.dtype),
        compiler_params=pltpu.CompilerParams(use_tc_tiling_on_sc=True),
        scratch_types=[
            pltpu.VMEM((chunk,), jnp.int32),
            pltpu.VMEM((chunk,), jnp.int32),
            pltpu.VMEM((chunk, d), table.dtype),
            pltpu.VMEM((chunk, d), table.dtype),
            pltpu.SemaphoreType.DMA,
            pltpu.SemaphoreType.DMA,
            pltpu.SemaphoreType.DMA,
        ],
    )
    def k(table_hbm, idx_hbm, out_hbm, idx_v0, idx_v1, rows_v0, rows_v1,
          sem_g, sem_o0, sem_o1):
        wid = lax.axis_index("s") * info.num_cores + lax.axis_index("c")
        base = pl.multiple_of(wid * per_w, 8)
        idx_v = (idx_v0, idx_v1)
        rows_v = (rows_v0, rows_v1)
        sem_o = (sem_o0, sem_o1)

        def chunk_step(ci, sl, guard):
            off = pl.multiple_of(base + ci * chunk, 8)

            @pl.when(guard)
            def _drain():
                pltpu.make_async_copy(
                    rows_v[sl], out_hbm.at[pl.ds(off, chunk)],
                    sem_o[sl]).wait()

            pltpu.sync_copy(idx_hbm.at[pl.ds(off, chunk)], idx_v[sl])
            pltpu.async_copy(table_hbm.at[idx_v[sl]], rows_v[sl],
                             sem_g).wait()
            pltpu.async_copy(rows_v[sl],
                             out_hbm.at[pl.ds(off, chunk)], sem_o[sl])

        def body(j, _):
            for sl in range(2):
                chunk_step(2 * j + sl, sl, j > 0)
            return ()

        lax.fori_loop(0, n_pairs, body, (), unroll=False)
        if tail:
            chunk_step(2 * n_pairs, 0, n_pairs > 0)
        for sl in range(2):
            if (n_pairs > 0) or (tail and sl == 0):
                pltpu.make_async_copy(rows_v[sl],
                                      out_hbm.at[pl.ds(base, chunk)],
                                      sem_o[sl]).wait()

    return k(table, idx_flat)


def _center_body(atom_ref, wc_ref, bf_ref, out_ref):
    out_ref[...] = (
        jnp.dot(atom_ref[...], wc_ref[...], preferred_element_type=jnp.float32)
        + bf_ref[...]
    )


def _edge_preact(g, q, wg, wn):
    return (
        jnp.dot(g.astype(jnp.bfloat16), wg, preferred_element_type=jnp.float32)
        + jnp.dot(q, wn, preferred_element_type=jnp.float32)
    )


def _stats_body(g_ref, q_ref, a_ref, wg_ref, wn_ref, prev_ref, stats_ref):
    step = pl.program_id(0)

    @pl.when(step == 0)
    def _init():
        stats_ref[...] = prev_ref[...]

    a = a_ref[...]
    q = q_ref[...].astype(jnp.bfloat16)
    x2 = _edge_preact(g_ref[...], q, wg_ref[...], wn_ref[...])
    ba, hf = a.shape
    m = x2.shape[0] // ba
    t = jnp.sum(x2.reshape(ba, m, hf), axis=1)
    s = jnp.sum(t, axis=0) + m * jnp.sum(a, axis=0)
    sq = (
        jnp.sum(x2 * x2, axis=0)
        + 2.0 * jnp.sum(a * t, axis=0)
        + m * jnp.sum(a * a, axis=0)
    )
    stats_ref[0:1, :] += s.reshape(1, hf)
    stats_ref[1:2, :] += sq.reshape(1, hf)


def _apply_body(count, g_ref, q_ref, a_ref, w_ref, wg_ref, wn_ref,
                stats_ref, g1_ref, b1_ref, prev2_ref, s_out_ref, st2_ref,
                wgs_scr, wns_scr):
    step = pl.program_id(0)

    @pl.when(step == 0)
    def _init_stats():
        st2_ref[...] = prev2_ref[...]

    mu = stats_ref[0:1, :] / count
    ex2 = stats_ref[1:2, :] / count
    var = ex2 - mu * mu
    inv = lax.rsqrt(var + _EPS)
    scale = g1_ref[...] * inv
    shift = b1_ref[...] - mu * scale

    @pl.when(step == 0)
    def _scale_weights():
        wgs_scr[...] = (wg_ref[...] * scale.astype(jnp.bfloat16))
        wns_scr[...] = (wn_ref[...] * scale.astype(jnp.bfloat16))

    q = q_ref[...].astype(jnp.bfloat16)
    y2 = _edge_preact(g_ref[...], q, wgs_scr[...], wns_scr[...])
    a = a_ref[...]
    ba, hf = a.shape
    m = y2.shape[0] // ba
    af = hf // 2
    a2 = a * scale + shift
    y3 = y2.reshape(ba, m, hf) + a2[:, None, :]
    f = _sigmoid(y3[:, :, :af])
    c = _softplus(y3[:, :, af:])
    w = w_ref[...]
    prod = f * c * (w * w)[:, :, None]
    s_blk = jnp.sum(prod, axis=1)
    s_out_ref[...] = s_blk
    st2_ref[0:1, :] += jnp.sum(s_blk, axis=0).reshape(1, af)
    st2_ref[1:2, :] += jnp.sum(s_blk * s_blk, axis=0).reshape(1, af)


def _final_body(count, hb2, sa_ref, sb_ref, atom_ref, st2_ref, g2_ref,
                b2_ref, wp_ref, bp_ref, out_ref):
    step = pl.program_id(0)
    mu = st2_ref[0:1, :] / count
    ex2 = st2_ref[1:2, :] / count
    var = ex2 - mu * mu
    inv = lax.rsqrt(var + _EPS)
    scale = g2_ref[...] * inv
    shift = b2_ref[...] - mu * scale
    s = jnp.where(step < hb2, sa_ref[...], sb_ref[...])
    h = _softplus(atom_ref[...] + s * scale + shift)
    out_ref[...] = (
        jnp.dot(h, wp_ref[...], preferred_element_type=jnp.float32)
        + bp_ref[...]
    )


def _tc_pipeline(atom_fea, g0a, g0b, nbr_flat, bond_w, W_full, b_full,
                 g1, b1, g2, b2, W_proj, b_proj):
    n, af = atom_fea.shape
    nm = 2 * g0a.shape[0]
    m = nm // n
    nf = nbr_flat.shape[1]
    hf = 2 * af

    wc = W_full[:af]
    wg = W_full[af:2 * af].astype(jnp.bfloat16)
    wn = W_full[2 * af:].astype(jnp.bfloat16)

    a_center = pl.pallas_call(
        _center_body,
        out_shape=jax.ShapeDtypeStruct((n, hf), jnp.float32),
    )(atom_fea, wc, b_full.reshape(1, hf))

    ba = 200
    be = ba * m
    half_n = n // 2
    hsteps = half_n // ba
    hblocks = half_n // ba
    full = lambda shp: pl.BlockSpec(shp, lambda i: (0,) * len(shp))
    zeros_hf = jnp.zeros((8, hf), jnp.float32)
    zeros_af = jnp.zeros((8, af), jnp.float32)

    def edge_specs(off):
        return [
            pl.BlockSpec((be, af), lambda i: (i, 0)),
            pl.BlockSpec((be, nf), lambda i: (i + off, 0)),
            pl.BlockSpec((ba, hf), lambda i: (i + off, 0)),
        ]

    def stats_call(g_half, off, prev):
        return pl.pallas_call(
            _stats_body,
            grid=(hsteps,),
            in_specs=edge_specs(off) + [full((af, hf)), full((nf, hf)),
                                        full((8, hf))],
            out_specs=pl.BlockSpec((8, hf), lambda i: (0, 0)),
            out_shape=jax.ShapeDtypeStruct((8, hf), jnp.float32),
        )(g_half, nbr_flat, a_center, wg, wn, prev)

    stats_a = stats_call(g0a, 0, zeros_hf)
    stats = stats_call(g0b, hblocks, stats_a)

    def apply_call(g_half, off, prev2):
        return pl.pallas_call(
            functools.partial(_apply_body, float(nm)),
            grid=(hsteps,),
            in_specs=edge_specs(off) + [
                pl.BlockSpec((ba, m), lambda i: (i + off, 0)),
                full((af, hf)),
                full((nf, hf)),
                full((8, hf)),
                full((1, hf)),
                full((1, hf)),
                full((8, af)),
            ],
            out_specs=[
                pl.BlockSpec((ba, af), lambda i: (i, 0)),
                pl.BlockSpec((8, af), lambda i: (0, 0)),
            ],
            out_shape=[
                jax.ShapeDtypeStruct((half_n, af), jnp.float32),
                jax.ShapeDtypeStruct((8, af), jnp.float32),
            ],
            scratch_shapes=[
                pltpu.VMEM((af, hf), jnp.bfloat16),
                pltpu.VMEM((nf, hf), jnp.bfloat16),
            ],
        )(g_half, nbr_flat, a_center, bond_w, wg, wn, stats,
          g1.reshape(1, hf), b1.reshape(1, hf), prev2)

    s_a, st2_a = apply_call(g0a, 0, zeros_af)
    s_b, st2 = apply_call(g0b, hblocks, st2_a)

    ba2 = 1000
    k4steps = n // ba2
    hb2 = half_n // ba2
    atom_out = pl.pallas_call(
        functools.partial(_final_body, float(n), hb2),
        grid=(k4steps,),
        in_specs=[
            pl.BlockSpec((ba2, af), lambda i: (jnp.minimum(i, hb2 - 1), 0)),
            pl.BlockSpec((ba2, af),
                         lambda i: (jnp.maximum(i - hb2, 0), 0)),
            pl.BlockSpec((ba2, af), lambda i: (i, 0)),
            pl.BlockSpec((8, af), lambda i: (0, 0)),
            pl.BlockSpec((1, af), lambda i: (0, 0)),
            pl.BlockSpec((1, af), lambda i: (0, 0)),
            pl.BlockSpec((af, af), lambda i: (0, 0)),
            pl.BlockSpec((1, af), lambda i: (0, 0)),
        ],
        out_specs=pl.BlockSpec((ba2, af), lambda i: (i, 0)),
        out_shape=jax.ShapeDtypeStruct((n, af), jnp.float32),
    )(s_a, s_b, atom_fea, st2, g2.reshape(1, af), b2.reshape(1, af),
      W_proj, b_proj.reshape(1, af))

    return atom_out


def kernel(atom_fea, nbr_fea, nbr_fea_idx, bond_weights_ag,
           W_full, b_full, g1, b1, g2, b2, W_proj, b_proj):
    n, m = nbr_fea_idx.shape
    nf = nbr_fea.shape[2]
    idx_flat = nbr_fea_idx.reshape(n * m).astype(jnp.int32)
    g0a = _sc_gather(atom_fea, idx_flat[: n * m // 2])
    g0b = _sc_gather(atom_fea, idx_flat[n * m // 2:])
    nbr_flat = nbr_fea.reshape(n * m, nf)
    atom_out = _tc_pipeline(atom_fea, g0a, g0b, nbr_flat, bond_weights_ag,
                            W_full, b_full, g1, b1, g2, b2, W_proj, b_proj)
    return atom_out, nbr_fea
